# baseline (speedup 1.0000x reference)
import functools

import jax
import jax.numpy as jnp
from jax import lax
from jax.experimental import pallas as pl
from jax.experimental.pallas import tpu as pltpu
from jax.experimental.pallas import tpu_sc as plsc

VOCAB = 1000000
D = 64
OUT = 1000
B = 4096
SEQ = 200

NC = 2
NS = 16
NW = NC * NS
B_PER_W = B // NW
HALF = SEQ // 2
HALVES_PER_W = 2 * B_PER_W

S = 8192
HS = S // 2
NF = VOCAB // S
TAILC = VOCAB - NF * S
HT = TAILC // 2
TB = NF * S


def _tc_pack(embT):
    G = 30
    REST = NF - 4 * G

    def body(in_hbm, o_hbm, sl0, sl1, sl2, sl3, ob0, ob1, tbuf,
             sin0, sin1, sin2, sin3, sout0, sout1, stail):
        g = pl.program_id(0)
        slabs = (sl0, sl1, sl2, sl3)
        sins = (sin0, sin1, sin2, sin3)
        obs = (ob0, ob1)
        souts = (sout0, sout1)

        def in_start(chunk, k):
            pltpu.make_async_copy(
                in_hbm.at[:, pl.ds(chunk * S, HS)],
                slabs[k].at[pl.ds(0, D)], sins[k]).start()
            pltpu.make_async_copy(
                in_hbm.at[:, pl.ds(chunk * S + HS, HS)],
                slabs[k].at[pl.ds(D, D)], sins[k]).start()

        def in_wait(k):
            pltpu.make_async_copy(
                in_hbm.at[:, pl.ds(0, HS)],
                slabs[k].at[pl.ds(0, D)], sins[k]).wait()
            pltpu.make_async_copy(
                in_hbm.at[:, pl.ds(0, HS)],
                slabs[k].at[pl.ds(D, D)], sins[k]).wait()

        def out_wait(m):
            pltpu.make_async_copy(
                obs[m], o_hbm.at[pl.ds(0, HS)], souts[m]).wait()

        def emit(k, chunk):
            m = k % 2
            obs[m][...] = jnp.swapaxes(slabs[k][...], 0, 1)
            pltpu.make_async_copy(
                obs[m], o_hbm.at[pl.ds(chunk * HS, HS)], souts[m]).start()

        @pl.when(g == 0)
        def _():
            in_start(0, 0)
            in_start(1, 1)
            in_start(2, 2)

        for k in range(4):
            c = 4 * g + k
            in_wait(k)
            if k == 0:
                in_start(c + 3, 3)
            else:
                @pl.when(g + 1 < G)
                def _():
                    in_start(c + 3, (k + 3) % 4)
            if k < 2:
                @pl.when(g > 0)
                def _():
                    out_wait(k % 2)
            else:
                out_wait(k % 2)
            emit(k, c)

        @pl.when(g == G - 1)
        def _():
            in_start(4 * G, 0)
            in_start(4 * G + 1, 1)
            in_wait(0)
            out_wait(0)
            emit(0, 4 * G)
            in_wait(1)
            out_wait(1)
            emit(1, 4 * G + 1)

            out_wait(0)
            t1 = pltpu.make_async_copy(
                in_hbm.at[:, pl.ds(TB, 512)], tbuf.at[:, pl.ds(0, 512)],
                stail)
            t1.start()
            t2 = pltpu.make_async_copy(
                in_hbm.at[:, pl.ds(TB + 512, TAILC - 512)],
                tbuf.at[:, pl.ds(512, TAILC - 512)], stail)
            t2.start()
            t1.wait()
            t2.wait()
            ob0[pl.ds(0, HT), pl.ds(0, D)] = (
                jnp.swapaxes(tbuf[:, pl.ds(0, HT)], 0, 1))
            ob0[pl.ds(0, HT), pl.ds(D, D)] = (
                jnp.swapaxes(tbuf[:, pl.ds(HT, HT)], 0, 1))
            to = pltpu.make_async_copy(
                ob0.at[pl.ds(0, HT)], o_hbm.at[pl.ds(NF * HS, HT)], stail)
            to.start()
            to.wait()
            out_wait(1)

    assert REST == 2
    return pl.pallas_call(
        body,
        grid=(G,),
        in_specs=[pl.BlockSpec(memory_space=pl.ANY)],
        out_specs=pl.BlockSpec(memory_space=pl.ANY),
        out_shape=jax.ShapeDtypeStruct((VOCAB // 2, 2 * D), jnp.float32),
        scratch_shapes=[
            pltpu.VMEM((2 * D, HS), jnp.float32),
            pltpu.VMEM((2 * D, HS), jnp.float32),
            pltpu.VMEM((2 * D, HS), jnp.float32),
            pltpu.VMEM((2 * D, HS), jnp.float32),
            pltpu.VMEM((HS, 2 * D), jnp.float32),
            pltpu.VMEM((HS, 2 * D), jnp.float32),
            pltpu.VMEM((D, TAILC), jnp.float32),
            pltpu.SemaphoreType.DMA,
            pltpu.SemaphoreType.DMA,
            pltpu.SemaphoreType.DMA,
            pltpu.SemaphoreType.DMA,
            pltpu.SemaphoreType.DMA,
            pltpu.SemaphoreType.DMA,
            pltpu.SemaphoreType.DMA,
        ],
    )(embT)


def _tc_xform(x2):
    def body(x_ref, o_ref):
        v = x_ref[...]
        t = jnp.bitwise_and(v, S - 1)
        h = (t >= HS).astype(jnp.int32)
        q_main = v + t - (S - 1) * h
        t2 = v - TB
        h2 = (t2 >= HT).astype(jnp.int32)
        q_tail = v + t2 - (TAILC - 1) * h2
        o_ref[...] = jnp.where(v >= TB, q_tail, q_main)

    return pl.pallas_call(
        body,
        grid=(8,),
        in_specs=[pl.BlockSpec((2 * B // 8, HALF), lambda i: (i, 0))],
        out_specs=pl.BlockSpec((2 * B // 8, HALF), lambda i: (i, 0)),
        out_shape=jax.ShapeDtypeStruct((2 * B, HALF), jnp.int32),
    )(x2)


def _sc_pool(x2, emb):
    mesh = plsc.VectorSubcoreMesh(core_axis_name="c", subcore_axis_name="s")

    @functools.partial(
        pl.kernel,
        out_type=jax.ShapeDtypeStruct((B, D), jnp.float32),
        mesh=mesh,
        compiler_params=pltpu.CompilerParams(use_tc_tiling_on_sc=False),
        scratch_types=(
            [pltpu.VMEM((HALVES_PER_W, HALF), jnp.int32)]
            + [pltpu.VMEM((HALF, D), jnp.float32)] * 8
            + [pltpu.VMEM((B_PER_W, D), jnp.float32)]
            + [pltpu.SemaphoreType.DMA] * 8
        ),
    )
    def k(x_hbm, emb_hbm, out_hbm, idx_v, r0, r1, r2, r3, r4, r5, r6, r7,
          pooled_v, s0, s1, s2, s3, s4, s5, s6, s7):
        wid = lax.axis_index("s") * NC + lax.axis_index("c")
        base_half = wid * HALVES_PER_W
        pltpu.sync_copy(x_hbm.at[pl.ds(base_half, HALVES_PER_W)], idx_v)

        bufs = (r0, r1, r2, r3, r4, r5, r6, r7)
        sems = (s0, s1, s2, s3, s4, s5, s6, s7)

        def fire(h, k):
            pltpu.async_copy(emb_hbm.at[idx_v.at[h]], bufs[k], sems[k])

        def wait(h, k):
            pltpu.make_async_copy(
                emb_hbm.at[idx_v.at[h]], bufs[k], sems[k]).wait()

        def accumulate(rows_ref, accs):
            def seq_body(s, accs):
                a0, a1, a2, a3 = accs
                return (a0 + rows_ref[s, pl.ds(0, 16)],
                        a1 + rows_ref[s, pl.ds(16, 16)],
                        a2 + rows_ref[s, pl.ds(32, 16)],
                        a3 + rows_ref[s, pl.ds(48, 16)])
            return lax.fori_loop(0, HALF, seq_body, accs, unroll=10)

        def store_row(i, accs):
            a0, a1, a2, a3 = accs
            pooled_v[i, pl.ds(0, 16)] = a0
            pooled_v[i, pl.ds(16, 16)] = a1
            pooled_v[i, pl.ds(32, 16)] = a2
            pooled_v[i, pl.ds(48, 16)] = a3

        for c in range(7):
            fire(c, c)

        def quad_body(j, _):
            base = 8 * j
            z = jnp.zeros((16,), jnp.float32)

            def step(c, k):
                wait(c, k)

                @pl.when(c + 7 < HALVES_PER_W)
                def _():
                    fire(c + 7, (k + 7) % 8)

            for r in range(4):
                c = base + 2 * r
                k0, k1 = 2 * r, 2 * r + 1
                step(c, k0)
                accs = accumulate(bufs[k0], (z, z, z, z))
                step(c + 1, k1)
                store_row(4 * j + r, accumulate(bufs[k1], accs))
            return 0

        lax.fori_loop(0, B_PER_W // 4, quad_body, 0)
        pltpu.sync_copy(pooled_v, out_hbm.at[pl.ds(wid * B_PER_W, B_PER_W)])

    return k(x2, emb)


def _tc_head(sums, wt, bcol):
    BLK = 512

    def body(p_ref, wt_ref, b_ref, o_ref):
        lt = lax.dot_general(
            wt_ref[...], p_ref[...], (((0,), (1,)), ((), ())),
            preferred_element_type=jnp.float32) + b_ref[...]
        m = jnp.max(lt, axis=0, keepdims=True)
        e = jnp.exp(lt - m)
        o_ref[...] = e / jnp.sum(e, axis=0, keepdims=True)

    return pl.pallas_call(
        body,
        grid=(B // BLK,),
        in_specs=[
            pl.BlockSpec((BLK, D), lambda i: (i, 0)),
            pl.BlockSpec((D, OUT), lambda i: (0, 0)),
            pl.BlockSpec((OUT, 1), lambda i: (0, 0)),
        ],
        out_specs=pl.BlockSpec((OUT, BLK), lambda i: (0, i)),
        out_shape=jax.ShapeDtypeStruct((OUT, B), jnp.float32),
    )(sums, wt, bcol)


def kernel(x, emb, W, b):
    xq = _tc_xform(x.reshape(2 * B, HALF).astype(jnp.int32))
    emb_lin = _tc_pack(emb.T).reshape(VOCAB, D)
    sums = _sc_pool(xq, emb_lin)
    wt = W.T * (1.0 / SEQ)
    return _tc_head(sums, wt, b.reshape(OUT, 1)).T


# speedup vs baseline: 1.0054x; 1.0054x over previous
import functools

import jax
import jax.numpy as jnp
from jax import lax
from jax.experimental import pallas as pl
from jax.experimental.pallas import tpu as pltpu
from jax.experimental.pallas import tpu_sc as plsc

VOCAB = 1000000
D = 64
OUT = 1000
B = 4096
SEQ = 200

NC = 2
NS = 16
NW = NC * NS
B_PER_W = B // NW
HALF = SEQ // 2
HALVES_PER_W = 2 * B_PER_W

S = 8192
HS = S // 2
NF = VOCAB // S
TAILC = VOCAB - NF * S
HT = TAILC // 2
TB = NF * S


def _tc_pack(embT):
    G = 30
    REST = NF - 4 * G

    def body(in_hbm, o_hbm, sl0, sl1, sl2, sl3, ob0, ob1, tbuf,
             sin0, sin1, sin2, sin3, sout0, sout1, stail):
        g = pl.program_id(0)
        slabs = (sl0, sl1, sl2, sl3)
        sins = (sin0, sin1, sin2, sin3)
        obs = (ob0, ob1)
        souts = (sout0, sout1)

        def in_start(chunk, k):
            pltpu.make_async_copy(
                in_hbm.at[:, pl.ds(chunk * S, HS)],
                slabs[k].at[pl.ds(0, D)], sins[k]).start()
            pltpu.make_async_copy(
                in_hbm.at[:, pl.ds(chunk * S + HS, HS)],
                slabs[k].at[pl.ds(D, D)], sins[k]).start()

        def in_wait(k):
            pltpu.make_async_copy(
                in_hbm.at[:, pl.ds(0, HS)],
                slabs[k].at[pl.ds(0, D)], sins[k]).wait()
            pltpu.make_async_copy(
                in_hbm.at[:, pl.ds(0, HS)],
                slabs[k].at[pl.ds(D, D)], sins[k]).wait()

        def out_wait(m):
            pltpu.make_async_copy(
                obs[m], o_hbm.at[pl.ds(0, HS)], souts[m]).wait()

        def emit(k, chunk):
            m = k % 2
            obs[m][...] = jnp.swapaxes(slabs[k][...], 0, 1)
            pltpu.make_async_copy(
                obs[m], o_hbm.at[pl.ds(chunk * HS, HS)], souts[m]).start()

        @pl.when(g == 0)
        def _():
            in_start(0, 0)
            in_start(1, 1)
            in_start(2, 2)

        for k in range(4):
            c = 4 * g + k
            in_wait(k)
            if k == 0:
                in_start(c + 3, 3)
            else:
                @pl.when(g + 1 < G)
                def _():
                    in_start(c + 3, (k + 3) % 4)
            if k < 2:
                @pl.when(g > 0)
                def _():
                    out_wait(k % 2)
            else:
                out_wait(k % 2)
            emit(k, c)

        @pl.when(g == G - 1)
        def _():
            in_start(4 * G, 0)
            in_start(4 * G + 1, 1)
            in_wait(0)
            out_wait(0)
            emit(0, 4 * G)
            in_wait(1)
            out_wait(1)
            emit(1, 4 * G + 1)

            out_wait(0)
            t1 = pltpu.make_async_copy(
                in_hbm.at[:, pl.ds(TB, 512)], tbuf.at[:, pl.ds(0, 512)],
                stail)
            t1.start()
            t2 = pltpu.make_async_copy(
                in_hbm.at[:, pl.ds(TB + 512, TAILC - 512)],
                tbuf.at[:, pl.ds(512, TAILC - 512)], stail)
            t2.start()
            t1.wait()
            t2.wait()
            ob0[pl.ds(0, HT), pl.ds(0, D)] = (
                jnp.swapaxes(tbuf[:, pl.ds(0, HT)], 0, 1))
            ob0[pl.ds(0, HT), pl.ds(D, D)] = (
                jnp.swapaxes(tbuf[:, pl.ds(HT, HT)], 0, 1))
            to = pltpu.make_async_copy(
                ob0.at[pl.ds(0, HT)], o_hbm.at[pl.ds(NF * HS, HT)], stail)
            to.start()
            to.wait()
            out_wait(1)

    assert REST == 2
    return pl.pallas_call(
        body,
        grid=(G,),
        in_specs=[pl.BlockSpec(memory_space=pl.ANY)],
        out_specs=pl.BlockSpec(memory_space=pl.ANY),
        out_shape=jax.ShapeDtypeStruct((VOCAB // 2, 2 * D), jnp.float32),
        scratch_shapes=[
            pltpu.VMEM((2 * D, HS), jnp.float32),
            pltpu.VMEM((2 * D, HS), jnp.float32),
            pltpu.VMEM((2 * D, HS), jnp.float32),
            pltpu.VMEM((2 * D, HS), jnp.float32),
            pltpu.VMEM((HS, 2 * D), jnp.float32),
            pltpu.VMEM((HS, 2 * D), jnp.float32),
            pltpu.VMEM((D, TAILC), jnp.float32),
            pltpu.SemaphoreType.DMA,
            pltpu.SemaphoreType.DMA,
            pltpu.SemaphoreType.DMA,
            pltpu.SemaphoreType.DMA,
            pltpu.SemaphoreType.DMA,
            pltpu.SemaphoreType.DMA,
            pltpu.SemaphoreType.DMA,
        ],
    )(embT)


def _tc_xform(x2):
    def body(x_ref, o_ref):
        v = x_ref[...]
        t = jnp.bitwise_and(v, S - 1)
        h = (t >= HS).astype(jnp.int32)
        q_main = v + t - (S - 1) * h
        t2 = v - TB
        h2 = (t2 >= HT).astype(jnp.int32)
        q_tail = v + t2 - (TAILC - 1) * h2
        o_ref[...] = jnp.where(v >= TB, q_tail, q_main)

    return pl.pallas_call(
        body,
        grid=(8,),
        in_specs=[pl.BlockSpec((2 * B // 8, HALF), lambda i: (i, 0))],
        out_specs=pl.BlockSpec((2 * B // 8, HALF), lambda i: (i, 0)),
        out_shape=jax.ShapeDtypeStruct((2 * B, HALF), jnp.int32),
    )(x2)


def _sc_pool(x2, emb):
    mesh = plsc.VectorSubcoreMesh(core_axis_name="c", subcore_axis_name="s")

    @functools.partial(
        pl.kernel,
        out_type=jax.ShapeDtypeStruct((B, D), jnp.float32),
        mesh=mesh,
        compiler_params=pltpu.CompilerParams(use_tc_tiling_on_sc=False),
        scratch_types=(
            [pltpu.VMEM((HALVES_PER_W, HALF), jnp.int32)]
            + [pltpu.VMEM((HALF, D), jnp.float32)] * 8
            + [pltpu.VMEM((B_PER_W, D), jnp.float32)]
            + [pltpu.SemaphoreType.DMA] * 8
        ),
    )
    def k(x_hbm, emb_hbm, out_hbm, idx_v, r0, r1, r2, r3, r4, r5, r6, r7,
          pooled_v, s0, s1, s2, s3, s4, s5, s6, s7):
        wid = lax.axis_index("s") * NC + lax.axis_index("c")
        base_half = wid * HALVES_PER_W
        pltpu.sync_copy(x_hbm.at[pl.ds(base_half, HALVES_PER_W)], idx_v)

        bufs = (r0, r1, r2, r3, r4, r5, r6, r7)
        sems = (s0, s1, s2, s3, s4, s5, s6, s7)

        def fire(h, k):
            pltpu.async_copy(emb_hbm.at[idx_v.at[h]], bufs[k], sems[k])

        def wait(h, k):
            pltpu.make_async_copy(
                emb_hbm.at[idx_v.at[h]], bufs[k], sems[k]).wait()

        def accumulate(rows_ref, accs):
            def seq_body(s, accs):
                a0, a1, a2, a3 = accs
                return (a0 + rows_ref[s, pl.ds(0, 16)],
                        a1 + rows_ref[s, pl.ds(16, 16)],
                        a2 + rows_ref[s, pl.ds(32, 16)],
                        a3 + rows_ref[s, pl.ds(48, 16)])
            return lax.fori_loop(0, HALF, seq_body, accs, unroll=4)

        def store_row(i, accs):
            a0, a1, a2, a3 = accs
            pooled_v[i, pl.ds(0, 16)] = a0
            pooled_v[i, pl.ds(16, 16)] = a1
            pooled_v[i, pl.ds(32, 16)] = a2
            pooled_v[i, pl.ds(48, 16)] = a3

        for c in range(7):
            fire(c, c)

        def quad_body(j, _):
            base = 8 * j
            z = jnp.zeros((16,), jnp.float32)

            def step(c, k):
                wait(c, k)

                @pl.when(c + 7 < HALVES_PER_W)
                def _():
                    fire(c + 7, (k + 7) % 8)

            for r in range(4):
                c = base + 2 * r
                k0, k1 = 2 * r, 2 * r + 1
                step(c, k0)
                accs = accumulate(bufs[k0], (z, z, z, z))
                step(c + 1, k1)
                store_row(4 * j + r, accumulate(bufs[k1], accs))
            return 0

        lax.fori_loop(0, B_PER_W // 4, quad_body, 0)
        pltpu.sync_copy(pooled_v, out_hbm.at[pl.ds(wid * B_PER_W, B_PER_W)])

    return k(x2, emb)


def _tc_head(sums, wt, bcol):
    BLK = 512

    def body(p_ref, wt_ref, b_ref, o_ref):
        lt = lax.dot_general(
            wt_ref[...], p_ref[...], (((0,), (1,)), ((), ())),
            preferred_element_type=jnp.float32) + b_ref[...]
        m = jnp.max(lt, axis=0, keepdims=True)
        e = jnp.exp(lt - m)
        o_ref[...] = e / jnp.sum(e, axis=0, keepdims=True)

    return pl.pallas_call(
        body,
        grid=(B // BLK,),
        in_specs=[
            pl.BlockSpec((BLK, D), lambda i: (i, 0)),
            pl.BlockSpec((D, OUT), lambda i: (0, 0)),
            pl.BlockSpec((OUT, 1), lambda i: (0, 0)),
        ],
        out_specs=pl.BlockSpec((OUT, BLK), lambda i: (0, i)),
        out_shape=jax.ShapeDtypeStruct((OUT, B), jnp.float32),
    )(sums, wt, bcol)


def kernel(x, emb, W, b):
    xq = _tc_xform(x.reshape(2 * B, HALF).astype(jnp.int32))
    emb_lin = _tc_pack(emb.T).reshape(VOCAB, D)
    sums = _sc_pool(xq, emb_lin)
    wt = W.T * (1.0 / SEQ)
    return _tc_head(sums, wt, b.reshape(OUT, 1)).T
